# baseline (speedup 1.0000x reference)
import functools

import jax
import jax.numpy as jnp
from jax import lax
from jax.experimental import pallas as pl
from jax.experimental.pallas import tpu as pltpu
from jax.experimental.pallas import tpu_sc as plsc

EMBED_NUM = 200
EMBED_DIM = 64
BATCH = 16384
HIST = 200

_NW = 16
_BPW = BATCH // _NW
_RPC = 2
_NCH = _BPW // _RPC
_NB = 2
_ITERS = _NCH // _NB
_SPLITS = ((0, 128), (128, 72))


def _sc_gather(idx, table):
    mesh = plsc.VectorSubcoreMesh(core_axis_name="c", subcore_axis_name="s", num_cores=1)

    @functools.partial(
        pl.kernel,
        mesh=mesh,
        out_type=jax.ShapeDtypeStruct((BATCH, HIST, EMBED_DIM), jnp.float32),
        scratch_types=[
            [pltpu.VMEM((_RPC, HIST), jnp.int32)] * _NB,
            [pltpu.VMEM((_RPC, HIST, EMBED_DIM), jnp.float32)] * _NB,
            pltpu.VMEM_SHARED((EMBED_NUM, EMBED_DIM), jnp.float32),
            [pltpu.SemaphoreType.DMA] * _NB,
            [pltpu.SemaphoreType.DMA] * _NB,
        ],
    )
    def k(idx_hbm, table_hbm, out_hbm, idx_v, rows_v, table_sp, gsem, ssem):
        wid = lax.axis_index("s")
        row0 = wid * _BPW

        @pl.when(lax.axis_index("s") == 0)
        def _():
            pltpu.sync_copy(table_hbm, table_sp)

        plsc.subcore_barrier()

        def fire(b, c):
            r = row0 + c * _RPC
            pltpu.sync_copy(idx_hbm.at[pl.ds(r, _RPC)], idx_v[b])
            for t in range(_RPC):
                for off, ln in _SPLITS:
                    pltpu.async_copy(
                        table_sp.at[idx_v[b].at[t].at[pl.ds(off, ln)]],
                        rows_v[b].at[t].at[pl.ds(off, ln)],
                        gsem[b],
                    )

        def wait_gather(b):
            for t in range(_RPC):
                for off, ln in _SPLITS:
                    pltpu.make_async_copy(
                        table_sp.at[idx_v[b].at[t].at[pl.ds(off, ln)]],
                        rows_v[b].at[t].at[pl.ds(off, ln)],
                        gsem[b],
                    ).wait()

        def out_slice(c):
            return out_hbm.at[pl.ds(row0 + c * _RPC, _RPC)]

        for b in range(_NB):
            fire(b, b)

        def body(i, _):
            for b in range(_NB):
                c = i * _NB + b
                wait_gather(b)
                pltpu.async_copy(rows_v[b], out_slice(c), ssem[b])
            for b in range(_NB):
                c = i * _NB + b
                pltpu.make_async_copy(rows_v[b], out_slice(c), ssem[b]).wait()

                @pl.when(c + _NB < _NCH)
                def _():
                    fire(b, c + _NB)

            return ()

        lax.fori_loop(0, _ITERS, body, (), unroll=False)

    return k(idx, table)


def kernel(visit_order, pos_embed_weight):
    return _sc_gather(visit_order.astype(jnp.int32), pos_embed_weight)


# speedup vs baseline: 1.0467x; 1.0467x over previous
import functools

import jax
from jax._src.pallas import mpmd as _mpmd
import jax.numpy as jnp
from jax import lax
from jax.experimental import pallas as pl
from jax.experimental.pallas import tpu as pltpu
from jax.experimental.pallas import tpu_sc as plsc

EMBED_NUM = 200
EMBED_DIM = 64
BATCH = 16384
HIST = 200

_NW = 32
_BPW = BATCH // _NW
_RPC = 2
_NCH = _BPW // _RPC
_NB = 2
_ITERS = _NCH // _NB
_SPLITS = ((0, 128), (128, 72))


def _sc_gather(idx, table):
    mesh = plsc.VectorSubcoreMesh(core_axis_name="c", subcore_axis_name="s")

    def k(dummy_hbm, idx_hbm, table_hbm, out_hbm, idx_v, rows_v, table_sp,
          gsem, ssem):
        wid = lax.axis_index("s") * 2 + lax.axis_index("c")
        row0 = wid * _BPW

        @pl.when(lax.axis_index("s") == 0)
        def _():
            pltpu.sync_copy(table_hbm, table_sp)

        plsc.subcore_barrier()

        def fire(b, c):
            r = row0 + c * _RPC
            pltpu.sync_copy(idx_hbm.at[pl.ds(r, _RPC)], idx_v[b])
            for t in range(_RPC):
                for off, ln in _SPLITS:
                    pltpu.async_copy(
                        table_sp.at[idx_v[b].at[t].at[pl.ds(off, ln)]],
                        rows_v[b].at[t].at[pl.ds(off, ln)],
                        gsem[b],
                    )

        def wait_gather(b):
            for t in range(_RPC):
                for off, ln in _SPLITS:
                    pltpu.make_async_copy(
                        table_sp.at[idx_v[b].at[t].at[pl.ds(off, ln)]],
                        rows_v[b].at[t].at[pl.ds(off, ln)],
                        gsem[b],
                    ).wait()

        def out_slice(c):
            return out_hbm.at[pl.ds(row0 + c * _RPC, _RPC)]

        for b in range(_NB):
            fire(b, b)

        def body(i, _):
            for b in range(_NB):
                c = i * _NB + b
                wait_gather(b)
                pltpu.async_copy(rows_v[b], out_slice(c), ssem[b])
            for b in range(_NB):
                c = i * _NB + b
                pltpu.make_async_copy(rows_v[b], out_slice(c), ssem[b]).wait()

                @pl.when(c + _NB < _NCH)
                def _():
                    fire(b, c + _NB)

            return ()

        lax.fori_loop(0, _ITERS, body, (), unroll=False)

    run = _mpmd._mpmd_map(
        [(mesh, k)],
        jax.ShapeDtypeStruct((BATCH, HIST, EMBED_DIM), jnp.float32),
        input_output_aliases={0: 0},
        scratch_types=[
            [pltpu.VMEM((_RPC, HIST), jnp.int32)] * _NB,
            [pltpu.VMEM((_RPC, HIST, EMBED_DIM), jnp.float32)] * _NB,
            pltpu.VMEM_SHARED((EMBED_NUM, EMBED_DIM), jnp.float32),
            [pltpu.SemaphoreType.DMA] * _NB,
            [pltpu.SemaphoreType.DMA] * _NB,
        ],
    )
    dummy = jnp.zeros((BATCH, HIST, EMBED_DIM), jnp.float32)
    return run(dummy, idx, table)


def kernel(visit_order, pos_embed_weight):
    return _sc_gather(visit_order.astype(jnp.int32), pos_embed_weight)


# speedup vs baseline: 1.6101x; 1.5383x over previous
import functools

import jax
import jax.numpy as jnp
from jax import lax
from jax.experimental import pallas as pl
from jax.experimental.pallas import tpu as pltpu
from jax.experimental.pallas import tpu_sc as plsc

EMBED_NUM = 200
EMBED_DIM = 64
BATCH = 16384
HIST = 200

_B = BATCH * HIST
_IDX_MINOR = 128
_IDX_ROWS = _B // _IDX_MINOR

_NW = 32
_ROWS_PER_W = _IDX_ROWS // _NW
_ROWS_PER_CH = 2
_CHUNK = _ROWS_PER_CH * _IDX_MINOR
_NCH = _ROWS_PER_W // _ROWS_PER_CH
_NB = 2
_ITERS = _NCH // _NB


def _sc_gather(idx2d, table):
    mesh = plsc.VectorSubcoreMesh(core_axis_name="c", subcore_axis_name="s")

    @functools.partial(
        pl.kernel,
        mesh=mesh,
        out_type=jax.ShapeDtypeStruct((_B, EMBED_DIM), jnp.float32),
        scratch_types=[
            [pltpu.VMEM((_ROWS_PER_CH, _IDX_MINOR), jnp.int32)] * _NB,
            [pltpu.VMEM((_CHUNK, EMBED_DIM), jnp.float32)] * _NB,
            pltpu.VMEM_SHARED((EMBED_NUM, EMBED_DIM), jnp.float32),
            [pltpu.SemaphoreType.DMA] * _NB,
            [pltpu.SemaphoreType.DMA] * _NB,
            [pltpu.SemaphoreType.DMA] * _NB,
        ],
    )
    def k(idx_hbm, table_hbm, out_hbm, idx_v, rows_v, table_sp, gsem, ssem,
          isem):
        wid = lax.axis_index("s") * 2 + lax.axis_index("c")
        row0 = wid * _ROWS_PER_W

        @pl.when(lax.axis_index("s") == 0)
        def _():
            pltpu.sync_copy(table_hbm, table_sp)

        plsc.subcore_barrier()

        def idx_copy(b, c):
            r = row0 + c * _ROWS_PER_CH
            return pltpu.make_async_copy(
                idx_hbm.at[pl.ds(r, _ROWS_PER_CH)], idx_v[b], isem[b]
            )

        def fire_gathers(b, c):
            idx_copy(b, c).wait()
            for j in range(_ROWS_PER_CH):
                pltpu.async_copy(
                    table_sp.at[idx_v[b].at[j]],
                    rows_v[b].at[pl.ds(j * _IDX_MINOR, _IDX_MINOR)],
                    gsem[b],
                )

        def wait_gather(b):
            for j in range(_ROWS_PER_CH):
                pltpu.make_async_copy(
                    table_sp.at[idx_v[b].at[j]],
                    rows_v[b].at[pl.ds(j * _IDX_MINOR, _IDX_MINOR)],
                    gsem[b],
                ).wait()

        def out_slice(c):
            return out_hbm.at[pl.ds((row0 + c * _ROWS_PER_CH) * _IDX_MINOR, _CHUNK)]

        for b in range(_NB):
            idx_copy(b, b).start()
        for b in range(_NB):
            fire_gathers(b, b)

        def body(i, _):
            for b in range(_NB):
                c = i * _NB + b
                wait_gather(b)

                @pl.when(c + _NB < _NCH)
                def _():
                    idx_copy(b, c + _NB).start()

                pltpu.async_copy(rows_v[b], out_slice(c), ssem[b])
            for b in range(_NB):
                c = i * _NB + b
                pltpu.make_async_copy(rows_v[b], out_slice(c), ssem[b]).wait()

                @pl.when(c + _NB < _NCH)
                def _():
                    fire_gathers(b, c + _NB)

            return ()

        lax.fori_loop(0, _ITERS, body, (), unroll=False)

    return k(idx2d, table)


def kernel(visit_order, pos_embed_weight):
    idx2d = jnp.reshape(visit_order.astype(jnp.int32), (_IDX_ROWS, _IDX_MINOR))
    flat = _sc_gather(idx2d, pos_embed_weight)
    return jnp.reshape(flat, (BATCH, HIST, EMBED_DIM))
